# Initial kernel scaffold; baseline (speedup 1.0000x reference)
import functools

import jax
import jax.numpy as jnp
from jax import lax
from jax.experimental import pallas as pl
from jax.experimental.pallas import tpu as pltpu
from jax.experimental.pallas import tpu_sc as plsc

N_NODES = 10000
N_EDGES = 320000
D_FEAT = 128
N_OUT = 7

NPAD = 10240
KF = 8
KH = 4
NW = 32
EPW = N_EDGES // NW


def _tc_project(x_pad, w_t):
    blk = 2048

    def body(x_ref, w_ref, out_ref):
        out_ref[...] = lax.dot_general(
            w_ref[...], x_ref[...],
            (((1,), (1,)), ((), ())),
            preferred_element_type=jnp.float32,
        )

    return pl.pallas_call(
        body,
        grid=(NPAD // blk,),
        in_specs=[
            pl.BlockSpec((blk, D_FEAT), lambda i: (i, 0)),
            pl.BlockSpec((KF, D_FEAT), lambda i: (0, 0)),
        ],
        out_specs=pl.BlockSpec((KF, blk), lambda i: (0, i)),
        out_shape=jax.ShapeDtypeStruct((KF, NPAD), jnp.float32),
    )(x_pad, w_t)


def _sc_scatter(xw_t, src, dst, wgt):
    mesh = plsc.VectorSubcoreMesh(core_axis_name="c", subcore_axis_name="s")

    @functools.partial(
        pl.kernel,
        mesh=mesh,
        out_type=jax.ShapeDtypeStruct((NW, KF, NPAD), jnp.float32),
        scratch_types=[
            pltpu.VMEM((KH, NPAD), jnp.float32),
            pltpu.VMEM((KH, NPAD), jnp.float32),
            pltpu.VMEM((EPW,), jnp.int32),
            pltpu.VMEM((EPW,), jnp.int32),
            pltpu.VMEM((EPW,), jnp.float32),
        ],
    )
    def sc_kernel(xwt_hbm, src_hbm, dst_hbm, wgt_hbm, out_hbm,
                  table_v, accum_v, src_v, dst_v, wgt_v):
        wid = lax.axis_index("c") * 16 + lax.axis_index("s")
        pltpu.sync_copy(src_hbm.at[wid], src_v)
        pltpu.sync_copy(dst_hbm.at[wid], dst_v)
        pltpu.sync_copy(wgt_hbm.at[wid], wgt_v)

        for p in range(KF // KH):
            pltpu.sync_copy(xwt_hbm.at[pl.ds(p * KH, KH)], table_v)

            def zero_body(i, carry):
                z = jnp.zeros((16,), jnp.float32)
                for k in range(KH):
                    accum_v[k, pl.ds(i * 16, 16)] = z
                return carry

            lax.fori_loop(0, NPAD // 16, zero_body, 0)

            def edge_body(j, carry):
                b = j * 16
                sv = src_v[pl.ds(b, 16)]
                dv = dst_v[pl.ds(b, 16)]
                wv = wgt_v[pl.ds(b, 16)]
                for k in range(KH):
                    ksp = jnp.full((16,), k, jnp.int32)
                    vals = plsc.load_gather(table_v, [ksp, sv])
                    plsc.addupdate_scatter(accum_v, [ksp, dv], vals * wv)
                return carry

            lax.fori_loop(0, EPW // 16, edge_body, 0)

            pltpu.sync_copy(accum_v, out_hbm.at[wid, pl.ds(p * KH, KH)])

    return sc_kernel(xw_t, src, dst, wgt)


def _tc_reduce_softmax(partials):
    blk = 1024

    def body(p_ref, out_ref):
        s = jnp.sum(p_ref[...], axis=0)
        valid = lax.broadcasted_iota(jnp.int32, (KF, blk), 0) < N_OUT
        m = jnp.max(jnp.where(valid, s, -jnp.inf), axis=0, keepdims=True)
        e = jnp.where(valid, jnp.exp(s - m), 0.0)
        out_ref[...] = e / jnp.sum(e, axis=0, keepdims=True)

    return pl.pallas_call(
        body,
        grid=(NPAD // blk,),
        in_specs=[pl.BlockSpec((NW, KF, blk), lambda i: (0, 0, i))],
        out_specs=pl.BlockSpec((KF, blk), lambda i: (0, i)),
        out_shape=jax.ShapeDtypeStruct((KF, NPAD), jnp.float32),
    )(partials)


def kernel(x, edge_index, edge_weight, W):
    x_pad = jnp.zeros((NPAD, D_FEAT), jnp.float32).at[:N_NODES].set(x)
    w_t = jnp.zeros((KF, D_FEAT), jnp.float32).at[:N_OUT].set(W.T)
    src = edge_index[0].astype(jnp.int32).reshape(NW, EPW)
    dst = edge_index[1].astype(jnp.int32).reshape(NW, EPW)
    wgt = edge_weight.reshape(NW, EPW)

    xw_t = _tc_project(x_pad, w_t)
    partials = _sc_scatter(xw_t, src, dst, wgt)
    sm = _tc_reduce_softmax(partials)
    return sm[:N_OUT, :N_NODES].T


# speedup vs baseline: 13.4712x; 13.4712x over previous
import functools

import jax
import jax.numpy as jnp
from jax import lax
from jax.experimental import pallas as pl
from jax.experimental.pallas import tpu as pltpu
from jax.experimental.pallas import tpu_sc as plsc

N_NODES = 10000
N_EDGES = 320000
D_FEAT = 128
N_OUT = 7

NPAD = 10240
KF = 8
KH = 4
NW = 32
EPW = N_EDGES // NW


def _tc_project(x_pad, w_t):
    blk = 2048

    def body(x_ref, w_ref, out_ref):
        out_ref[...] = lax.dot_general(
            w_ref[...], x_ref[...],
            (((1,), (1,)), ((), ())),
            preferred_element_type=jnp.float32,
        )

    return pl.pallas_call(
        body,
        grid=(NPAD // blk,),
        in_specs=[
            pl.BlockSpec((blk, D_FEAT), lambda i: (i, 0)),
            pl.BlockSpec((KF, D_FEAT), lambda i: (0, 0)),
        ],
        out_specs=pl.BlockSpec((KF, blk), lambda i: (0, i)),
        out_shape=jax.ShapeDtypeStruct((KF, NPAD), jnp.float32),
    )(x_pad, w_t)


def _sc_scatter(xw_t, src, dst, wgt):
    mesh = plsc.VectorSubcoreMesh(core_axis_name="c", subcore_axis_name="s")

    half = KH * NPAD

    @functools.partial(
        pl.kernel,
        mesh=mesh,
        out_type=jax.ShapeDtypeStruct((NW, KF * NPAD), jnp.float32),
        scratch_types=[
            pltpu.VMEM((half,), jnp.float32),
            pltpu.VMEM((half,), jnp.float32),
            pltpu.VMEM((EPW,), jnp.int32),
            pltpu.VMEM((EPW,), jnp.int32),
            pltpu.VMEM((EPW,), jnp.float32),
        ],
        compiler_params=pltpu.CompilerParams(needs_layout_passes=False),
    )
    def sc_kernel(xwt_hbm, src_hbm, dst_hbm, wgt_hbm, out_hbm,
                  table_v, accum_v, src_v, dst_v, wgt_v):
        wid = lax.axis_index("c") * 16 + lax.axis_index("s")
        pltpu.sync_copy(src_hbm.at[wid], src_v)
        pltpu.sync_copy(dst_hbm.at[wid], dst_v)
        pltpu.sync_copy(wgt_hbm.at[wid], wgt_v)

        for p in range(KF // KH):
            pltpu.sync_copy(xwt_hbm.at[pl.ds(p * half, half)], table_v)

            def zero_body(i, carry):
                accum_v[pl.ds(i * 16, 16)] = jnp.zeros((16,), jnp.float32)
                return carry

            lax.fori_loop(0, half // 16, zero_body, 0)

            def edge_body(j, carry):
                b = j * 16
                sv = src_v[pl.ds(b, 16)]
                dv = dst_v[pl.ds(b, 16)]
                wv = wgt_v[pl.ds(b, 16)]
                for k in range(KH):
                    vals = plsc.load_gather(table_v, [sv + (k * NPAD)])
                    plsc.addupdate_scatter(accum_v, [dv + (k * NPAD)], vals * wv)
                return carry

            lax.fori_loop(0, EPW // 16, edge_body, 0)

            pltpu.sync_copy(accum_v, out_hbm.at[wid, pl.ds(p * half, half)])

    return sc_kernel(xw_t.reshape(KF * NPAD), src, dst, wgt).reshape(NW, KF, NPAD)


def _tc_reduce_softmax(partials):
    blk = 1024

    def body(p_ref, out_ref):
        s = jnp.sum(p_ref[...], axis=0)
        valid = lax.broadcasted_iota(jnp.int32, (KF, blk), 0) < N_OUT
        m = jnp.max(jnp.where(valid, s, -jnp.inf), axis=0, keepdims=True)
        e = jnp.where(valid, jnp.exp(s - m), 0.0)
        out_ref[...] = e / jnp.sum(e, axis=0, keepdims=True)

    return pl.pallas_call(
        body,
        grid=(NPAD // blk,),
        in_specs=[pl.BlockSpec((NW, KF, blk), lambda i: (0, 0, i))],
        out_specs=pl.BlockSpec((KF, blk), lambda i: (0, i)),
        out_shape=jax.ShapeDtypeStruct((KF, NPAD), jnp.float32),
    )(partials)


def kernel(x, edge_index, edge_weight, W):
    x_pad = jnp.zeros((NPAD, D_FEAT), jnp.float32).at[:N_NODES].set(x)
    w_t = jnp.zeros((KF, D_FEAT), jnp.float32).at[:N_OUT].set(W.T)
    src = edge_index[0].astype(jnp.int32).reshape(NW, EPW)
    dst = edge_index[1].astype(jnp.int32).reshape(NW, EPW)
    wgt = edge_weight.reshape(NW, EPW)

    xw_t = _tc_project(x_pad, w_t)
    partials = _sc_scatter(xw_t, src, dst, wgt)
    sm = _tc_reduce_softmax(partials)
    return sm[:N_OUT, :N_NODES].T


# speedup vs baseline: 15.4282x; 1.1453x over previous
import functools

import jax
import jax.numpy as jnp
from jax import lax
from jax.experimental import pallas as pl
from jax.experimental.pallas import tpu as pltpu
from jax.experimental.pallas import tpu_sc as plsc

N_NODES = 10000
N_EDGES = 320000
D_FEAT = 128
N_OUT = 7

NPAD = 10240
KF = 8
KH = 4
NW = 32
EPW = N_EDGES // NW


def _tc_project(x_pad, w_t):
    blk = 2048

    def body(x_ref, w_ref, out_ref):
        out_ref[...] = lax.dot_general(
            w_ref[...], x_ref[...],
            (((1,), (1,)), ((), ())),
            preferred_element_type=jnp.float32,
        )

    return pl.pallas_call(
        body,
        grid=(NPAD // blk,),
        in_specs=[
            pl.BlockSpec((blk, D_FEAT), lambda i: (i, 0)),
            pl.BlockSpec((KF, D_FEAT), lambda i: (0, 0)),
        ],
        out_specs=pl.BlockSpec((KF, blk), lambda i: (0, i)),
        out_shape=jax.ShapeDtypeStruct((KF, NPAD), jnp.float32),
    )(x_pad, w_t)


def _sc_scatter(xw_t, src, dst, wgt):
    mesh = plsc.VectorSubcoreMesh(core_axis_name="c", subcore_axis_name="s")

    half = KH * NPAD

    @functools.partial(
        pl.kernel,
        mesh=mesh,
        out_type=jax.ShapeDtypeStruct((NW, KF * NPAD), jnp.float32),
        scratch_types=[
            pltpu.VMEM((half,), jnp.float32),
            pltpu.VMEM((half,), jnp.float32),
            pltpu.VMEM((EPW,), jnp.int32),
            pltpu.VMEM((EPW,), jnp.int32),
            pltpu.VMEM((EPW,), jnp.float32),
        ],
        compiler_params=pltpu.CompilerParams(needs_layout_passes=False),
    )
    def sc_kernel(xwt_hbm, src_hbm, dst_hbm, wgt_hbm, out_hbm,
                  table_v, accum_v, src_v, dst_v, wgt_v):
        wid = lax.axis_index("c") * 16 + lax.axis_index("s")
        pltpu.sync_copy(src_hbm.at[wid], src_v)
        pltpu.sync_copy(dst_hbm.at[wid], dst_v)
        pltpu.sync_copy(wgt_hbm.at[wid], wgt_v)

        for p in range(KF // KH):
            pltpu.sync_copy(xwt_hbm.at[pl.ds(p * half, half)], table_v)

            def zero_body(i, carry):
                accum_v[pl.ds(i * 16, 16)] = jnp.zeros((16,), jnp.float32)
                return carry

            lax.fori_loop(0, half // 16, zero_body, 0, unroll=8)

            def edge_body(j, carry):
                for g in range(4):
                    b = j * 64 + g * 16
                    sv = src_v[pl.ds(b, 16)]
                    dv = dst_v[pl.ds(b, 16)]
                    wv = wgt_v[pl.ds(b, 16)]
                    for k in range(KH):
                        vals = plsc.load_gather(table_v, [sv + (k * NPAD)])
                        plsc.addupdate_scatter(accum_v, [dv + (k * NPAD)], vals * wv)
                return carry

            lax.fori_loop(0, EPW // 64, edge_body, 0)

            pltpu.sync_copy(accum_v, out_hbm.at[wid, pl.ds(p * half, half)])

    return sc_kernel(xw_t.reshape(KF * NPAD), src, dst, wgt).reshape(NW, KF, NPAD)


def _tc_reduce_softmax(partials):
    blk = 1024

    def body(p_ref, out_ref):
        s = jnp.sum(p_ref[...], axis=0)
        valid = lax.broadcasted_iota(jnp.int32, (KF, blk), 0) < N_OUT
        m = jnp.max(jnp.where(valid, s, -jnp.inf), axis=0, keepdims=True)
        e = jnp.where(valid, jnp.exp(s - m), 0.0)
        out_ref[...] = e / jnp.sum(e, axis=0, keepdims=True)

    return pl.pallas_call(
        body,
        grid=(NPAD // blk,),
        in_specs=[pl.BlockSpec((NW, KF, blk), lambda i: (0, 0, i))],
        out_specs=pl.BlockSpec((KF, blk), lambda i: (0, i)),
        out_shape=jax.ShapeDtypeStruct((KF, NPAD), jnp.float32),
    )(partials)


def kernel(x, edge_index, edge_weight, W):
    x_pad = jnp.zeros((NPAD, D_FEAT), jnp.float32).at[:N_NODES].set(x)
    w_t = jnp.zeros((KF, D_FEAT), jnp.float32).at[:N_OUT].set(W.T)
    src = edge_index[0].astype(jnp.int32).reshape(NW, EPW)
    dst = edge_index[1].astype(jnp.int32).reshape(NW, EPW)
    wgt = edge_weight.reshape(NW, EPW)

    xw_t = _tc_project(x_pad, w_t)
    partials = _sc_scatter(xw_t, src, dst, wgt)
    sm = _tc_reduce_softmax(partials)
    return sm[:N_OUT, :N_NODES].T
